# Initial kernel scaffold; baseline (speedup 1.0000x reference)
import functools

import jax
import jax.numpy as jnp
from jax import lax
from jax.experimental import pallas as pl
from jax.experimental.pallas import tpu as pltpu
from jax.experimental.pallas import tpu_sc as plsc

EMB = 64
NC = 2
NS = 16
NW = NC * NS
CHUNK = 128


def _gather_body(t_hbm, table_hbm, out_hbm, idx_v, rows_v, sem):
    wid = lax.axis_index("s") * NC + lax.axis_index("c")
    n_chunks = t_hbm.shape[1]
    pltpu.sync_copy(t_hbm.at[wid], idx_v)

    def chunk(g, carry):
        pltpu.async_copy(table_hbm.at[idx_v.at[g]], rows_v, sem).wait()
        pltpu.sync_copy(rows_v, out_hbm.at[wid, g])
        return carry

    lax.fori_loop(0, n_chunks, chunk, 0, unroll=False)


def kernel(t, pos_embeddings):
    B, S = t.shape
    total = B * S
    assert total % (NW * CHUNK) == 0
    n_chunks = total // (NW * CHUNK)

    t_split = t.reshape(NW, n_chunks, CHUNK)
    mesh = plsc.VectorSubcoreMesh(core_axis_name="c", subcore_axis_name="s")

    run = functools.partial(
        pl.kernel,
        out_type=jax.ShapeDtypeStruct((NW, n_chunks, CHUNK, EMB), jnp.float32),
        mesh=mesh,
        scratch_types=[
            pltpu.VMEM((n_chunks, CHUNK), jnp.int32),
            pltpu.VMEM((CHUNK, EMB), jnp.float32),
            pltpu.SemaphoreType.DMA,
        ],
    )(_gather_body)

    out = run(t_split, pos_embeddings)
    return out.reshape(B, S, EMB)


# speedup vs baseline: 5.2156x; 5.2156x over previous
import functools

import jax
import jax.numpy as jnp
from jax import lax
from jax.experimental import pallas as pl
from jax.experimental.pallas import tpu as pltpu
from jax.experimental.pallas import tpu_sc as plsc

EMB = 64
NC = 2
NS = 16
NW = NC * NS
CHUNK = 128


def _gather_body(t_hbm, table_hbm, out_hbm, idx_v, rows_v, sem):
    wid = lax.axis_index("s") * NC + lax.axis_index("c")
    n_chunks = t_hbm.shape[1]
    pltpu.sync_copy(t_hbm.at[wid], idx_v)

    def chunk(g, carry):
        pltpu.async_copy(table_hbm.at[idx_v.at[g]], rows_v, sem).wait()
        pltpu.sync_copy(rows_v, out_hbm.at[wid, g])
        return carry

    lax.fori_loop(0, n_chunks, chunk, 0, unroll=False)


def kernel(t, pos_embeddings):
    B, S = t.shape
    total = B * S
    assert total % (NW * CHUNK) == 0
    n_chunks = total // (NW * CHUNK)

    t_split = t.reshape(NW, n_chunks, CHUNK)
    mesh = plsc.VectorSubcoreMesh(core_axis_name="c", subcore_axis_name="s")

    run = functools.partial(
        pl.kernel,
        out_type=jax.ShapeDtypeStruct((NW, n_chunks, CHUNK, EMB), jnp.float32),
        mesh=mesh,
        scratch_types=[
            pltpu.VMEM((n_chunks, CHUNK), jnp.int32),
            pltpu.VMEM((CHUNK, EMB), jnp.float32),
            pltpu.SemaphoreType.DMA,
        ],
        compiler_params=pltpu.CompilerParams(use_tc_tiling_on_sc=False),
    )(_gather_body)

    out = run(t_split, pos_embeddings)
    return out.reshape(B, S, EMB)


# speedup vs baseline: 6.2562x; 1.1995x over previous
import functools

import jax
import jax.numpy as jnp
from jax import lax
from jax.experimental import pallas as pl
from jax.experimental.pallas import tpu as pltpu
from jax.experimental.pallas import tpu_sc as plsc

EMB = 64
NC = 2
NS = 16
NW = NC * NS
CHUNK = 128
NBUF = 8
D = 4


def _gather_body(t_hbm, table_hbm, out_hbm, idx_v, rows_v, gsem, ssem):
    wid = lax.axis_index("s") * NC + lax.axis_index("c")
    n_chunks = t_hbm.shape[1]
    pltpu.sync_copy(t_hbm.at[wid], idx_v)

    def gather(g, slot):
        return pltpu.make_async_copy(
            table_hbm.at[idx_v.at[g]], rows_v.at[slot], gsem.at[slot])

    def store(g, slot):
        return pltpu.make_async_copy(
            rows_v.at[slot], out_hbm.at[wid, g], ssem.at[slot])

    for b in range(D):
        gather(b, b).start()

    def body(g, carry):
        slot = lax.rem(g, NBUF)
        gather(g, slot).wait()
        store(g, slot).start()
        nxt = g + D
        nslot = lax.rem(nxt, NBUF)

        @pl.when(nxt < n_chunks)
        def _():
            @pl.when(nxt >= NBUF)
            def _():
                store(nxt - NBUF, nslot).wait()

            gather(nxt, nslot).start()

        return carry

    lax.fori_loop(0, n_chunks, body, 0, unroll=False)

    for b in range(NBUF):
        g = n_chunks - NBUF + b
        store(g, g % NBUF).wait()


def kernel(t, pos_embeddings):
    B, S = t.shape
    total = B * S
    assert total % (NW * CHUNK) == 0
    n_chunks = total // (NW * CHUNK)

    t_split = t.reshape(NW, n_chunks, CHUNK)
    mesh = plsc.VectorSubcoreMesh(core_axis_name="c", subcore_axis_name="s")

    run = functools.partial(
        pl.kernel,
        out_type=jax.ShapeDtypeStruct((NW, n_chunks, CHUNK, EMB), jnp.float32),
        mesh=mesh,
        scratch_types=[
            pltpu.VMEM((n_chunks, CHUNK), jnp.int32),
            pltpu.VMEM((NBUF, CHUNK, EMB), jnp.float32),
            pltpu.SemaphoreType.DMA((NBUF,)),
            pltpu.SemaphoreType.DMA((NBUF,)),
        ],
        compiler_params=pltpu.CompilerParams(use_tc_tiling_on_sc=False),
    )(_gather_body)

    out = run(t_split, pos_embeddings)
    return out.reshape(B, S, EMB)
